# baseline (speedup 1.0000x reference)
import functools

import jax
import jax.numpy as jnp
from jax import lax
from jax.experimental import pallas as pl
from jax.experimental.pallas import tpu as pltpu
from jax.experimental.pallas import tpu_sc as plsc

B = 4096
D = 1024
K = 4
KP = 8
KM_ITERS = 10


def _feat_body(x_ref, we_ref, be_ref, f_ref):
    f_ref[...] = jnp.maximum(
        jnp.dot(x_ref[...], we_ref[...], preferred_element_type=jnp.float32)
        + be_ref[...],
        0.0,
    )


_FM = 512


def _tc_feat(x, W_ext, b_ext2):
    return pl.pallas_call(
        _feat_body,
        grid=(B // _FM,),
        in_specs=[
            pl.BlockSpec((_FM, D), lambda i: (i, 0)),
            pl.BlockSpec((D, D), lambda i: (0, 0)),
            pl.BlockSpec((1, D), lambda i: (0, 0)),
        ],
        out_specs=pl.BlockSpec((_FM, D), lambda i: (i, 0)),
        out_shape=jax.ShapeDtypeStruct((B, D), jnp.float32),
    )(x, W_ext, b_ext2)


_RC = 128


def _km_body(f_ref, lab_ref, rank_ref, oh_scr):
    f = f_ref[...]
    fsq = jnp.sum(f * f, axis=1, keepdims=True)
    col_k = lax.broadcasted_iota(jnp.int32, (1, KP), 1)
    pad_mask = jnp.where(col_k >= K, jnp.float32(1e30), jnp.float32(0.0))
    oh_iota = lax.broadcasted_iota(jnp.int32, (B, KP), 1)

    def km_iter(_, carry):
        c, _ = carry
        d2 = (
            fsq
            - 2.0 * lax.dot_general(f, c, (((1,), (1,)), ((), ())),
                                    preferred_element_type=jnp.float32)
            + jnp.sum(c * c, axis=1)[None, :]
            + pad_mask
        )
        labels = jnp.argmin(d2, axis=1).astype(jnp.int32)
        oh = (labels[:, None] == oh_iota).astype(jnp.float32)
        sums = lax.dot_general(oh, f, (((0,), (0,)), ((), ())),
                               preferred_element_type=jnp.float32)
        counts = jnp.maximum(jnp.sum(oh, axis=0)[:, None], 1.0)
        return sums / counts, labels

    c0 = f[0:KP]
    _, labels = lax.fori_loop(
        0, KM_ITERS, km_iter, (c0, jnp.zeros((B,), jnp.int32))
    )

    oh = (labels[:, None] == oh_iota).astype(jnp.float32)
    oh_scr[...] = oh
    lab_ref[...] = labels[:, None]

    counts_row = jnp.sum(oh, axis=0, keepdims=True)
    s_lo = lax.broadcasted_iota(jnp.int32, (KP, KP), 0)
    s_hi = lax.broadcasted_iota(jnp.int32, (KP, KP), 1)
    strict = (s_lo < s_hi).astype(jnp.float32)
    offs_g = lax.dot_general(counts_row, strict, (((1,), (0,)), ((), ())),
                             preferred_element_type=jnp.float32,
                             precision=lax.Precision.HIGHEST)
    l_lo = lax.broadcasted_iota(jnp.int32, (_RC, _RC), 0)
    l_hi = lax.broadcasted_iota(jnp.int32, (_RC, _RC), 1)
    tril = (l_lo >= l_hi).astype(jnp.float32)

    def chunk_body(i, tot):
        oh_c = oh_scr[pl.ds(i * _RC, _RC), :]
        incl = lax.dot_general(tril, oh_c, (((1,), (0,)), ((), ())),
                               preferred_element_type=jnp.float32,
                               precision=lax.Precision.HIGHEST) + tot
        r = jnp.sum(oh_c * (incl - 1.0 + offs_g), axis=1, keepdims=True)
        rank_ref[pl.ds(i * _RC, _RC), :] = r.astype(jnp.int32)
        return tot + jnp.sum(oh_c, axis=0, keepdims=True)

    lax.fori_loop(0, B // _RC, chunk_body, jnp.zeros((1, KP), jnp.float32))


def _tc_kmeans(feat):
    return pl.pallas_call(
        _km_body,
        out_shape=[
            jax.ShapeDtypeStruct((B, 1), jnp.int32),
            jax.ShapeDtypeStruct((B, 1), jnp.int32),
        ],
        scratch_shapes=[pltpu.VMEM((B, KP), jnp.float32)],
    )(feat)


def _est_body(f_ref, lab_ref, ws_ref, bs_ref, pe_ref, z_ref):
    pe_proj = jnp.dot(pe_ref[...], ws_ref[...],
                      preferred_element_type=jnp.float32)
    lab_blk = lab_ref[...]
    oh_blk = (lab_blk
              == lax.broadcasted_iota(jnp.int32, (1, K), 1)).astype(jnp.float32)
    pe_add = lax.dot_general(oh_blk, pe_proj, (((1,), (0,)), ((), ())),
                             preferred_element_type=jnp.float32)
    z_ref[...] = (
        jnp.dot(f_ref[...], ws_ref[...], preferred_element_type=jnp.float32)
        + bs_ref[...]
        + pe_add
    )


def _tc_est(feat, labels_row, W_est, b_est2, pe_table):
    return pl.pallas_call(
        _est_body,
        grid=(B // _FM,),
        in_specs=[
            pl.BlockSpec((_FM, D), lambda i: (i, 0)),
            pl.BlockSpec((_FM, 1), lambda i: (i, 0)),
            pl.BlockSpec((D, D), lambda i: (0, 0)),
            pl.BlockSpec((1, D), lambda i: (0, 0)),
            pl.BlockSpec((K, D), lambda i: (0, 0)),
        ],
        out_specs=pl.BlockSpec((_FM, D), lambda i: (i, 0)),
        out_shape=jax.ShapeDtypeStruct((B, D), jnp.float32),
    )(feat, labels_row, W_est, b_est2, pe_table)


_NW = 32
_PER = B // _NW
_CHUNK = 64


def _sc_permute(z, rank):
    mesh = plsc.VectorSubcoreMesh(core_axis_name="c", subcore_axis_name="s")

    @functools.partial(
        pl.kernel,
        out_type=jax.ShapeDtypeStruct((B, D), jnp.float32),
        mesh=mesh,
        scratch_types=[
            pltpu.VMEM((_CHUNK,), jnp.int32),
            pltpu.VMEM((_CHUNK, D), jnp.float32),
            pltpu.SemaphoreType.DMA,
        ],
    )
    def k(z_hbm, rank_hbm, out_hbm, idx_v, rows_v, sem):
        wid = lax.axis_index("s") * 2 + lax.axis_index("c")
        base = wid * _PER
        for c in range(_PER // _CHUNK):
            off = base + c * _CHUNK
            pltpu.sync_copy(rank_hbm.at[pl.ds(off, _CHUNK)], idx_v)
            pltpu.sync_copy(z_hbm.at[pl.ds(off, _CHUNK)], rows_v)
            pltpu.async_copy(rows_v, out_hbm.at[idx_v], sem).wait()

    return k(z, rank)


def kernel(x, W_ext, b_ext, W_est, b_est, pe_table):
    feat = _tc_feat(x, W_ext, b_ext.reshape(1, D))
    labels_col, rank_col = _tc_kmeans(feat)
    z = _tc_est(feat, labels_col, W_est, b_est.reshape(1, D), pe_table)
    return _sc_permute(z, rank_col.reshape(B))
